# baseline (speedup 1.0000x reference)
import functools

import jax
import jax.numpy as jnp
from jax import lax
from jax.experimental import pallas as pl
from jax.experimental.pallas import tpu as pltpu
from jax.experimental.pallas import tpu_sc as plsc

GAMMA = 0.1
EPSILON = 0.1
N, E, D, OUT = 10000, 320000, 128, 40

NC, NS = 2, 16
NW = NC * NS
EPW = E // NW
CHUNK = 80
NCHUNK = EPW // CHUNK
NPADH = 10240
HSTRIPE = NPADH // NS
NPADR = 10240
RSTRIPE = NPADR // NS

def _hist_body(row3_hbm, out_hbm, idx_all_v, ones_v, zb_v, hist_sh, sem):
    cid = lax.axis_index("core")
    sid = lax.axis_index("subcore")
    wid = sid * NC + cid

    @pl.loop(0, HSTRIPE // 16)
    def _(i):
        zb_v[pl.ds(i * 16, 16)] = jnp.zeros((16,), jnp.float32)

    @pl.loop(0, CHUNK // 16)
    def _(i):
        ones_v[pl.ds(i * 16, 16)] = jnp.ones((16,), jnp.float32)

    pltpu.async_copy(row3_hbm.at[wid], idx_all_v, sem)
    pltpu.sync_copy(zb_v, hist_sh.at[pl.ds(sid * HSTRIPE, HSTRIPE)])
    pltpu.make_async_copy(row3_hbm.at[wid], idx_all_v, sem).wait()
    plsc.subcore_barrier()

    @pl.loop(0, NCHUNK)
    def _(c):
        pltpu.sync_copy(ones_v, hist_sh.at[idx_all_v.at[c]], add=True)

    plsc.subcore_barrier()
    pltpu.sync_copy(hist_sh.at[pl.ds(sid * HSTRIPE, HSTRIPE)],
                    out_hbm.at[cid, pl.ds(sid * HSTRIPE, HSTRIPE)])


@functools.cache
def _get_hist():
    mesh = plsc.VectorSubcoreMesh(core_axis_name="core",
                                  subcore_axis_name="subcore")
    return pl.kernel(
        _hist_body,
        out_type=jax.ShapeDtypeStruct((NC, NPADH), jnp.float32),
        mesh=mesh,
        scratch_types=[
            pltpu.VMEM((NCHUNK, CHUNK), jnp.int32),
            pltpu.VMEM((CHUNK,), jnp.float32),
            pltpu.VMEM((HSTRIPE,), jnp.float32),
            pltpu.VMEM_SHARED((NPADH,), jnp.float32),
            pltpu.SemaphoreType.DMA,
        ],
    )


def _scatter_body(y_hbm, row_hbm, col_hbm, out_hbm, ridx_all_v, cidx_all_v,
                  rows_a, rows_b, acc_sh, sem_a, sem_b):
    cid = lax.axis_index("core")
    sid = lax.axis_index("subcore")
    wid = sid * NC + cid

    @pl.loop(0, CHUNK)
    def _(r):
        @pl.loop(0, D // 16)
        def _(c):
            rows_a[r, pl.ds(c * 16, 16)] = jnp.zeros((16,), jnp.float32)

    pltpu.async_copy(row_hbm.at[pl.ds(wid * EPW, EPW)], ridx_all_v, sem_b)
    pltpu.async_copy(col_hbm.at[pl.ds(wid * EPW, EPW)], cidx_all_v, sem_b)

    @pl.loop(0, RSTRIPE // CHUNK)
    def _(j):
        pltpu.async_copy(rows_a, acc_sh.at[pl.ds(sid * RSTRIPE + j * CHUNK,
                                                 CHUNK)], sem_a)

    @pl.loop(0, RSTRIPE // CHUNK)
    def _(j):
        pltpu.make_async_copy(rows_a,
                              acc_sh.at[pl.ds(sid * RSTRIPE + j * CHUNK,
                                              CHUNK)], sem_a).wait()
    pltpu.make_async_copy(row_hbm.at[pl.ds(wid * EPW, EPW)], ridx_all_v,
                          sem_b).wait()
    pltpu.make_async_copy(col_hbm.at[pl.ds(wid * EPW, EPW)], cidx_all_v,
                          sem_b).wait()
    plsc.subcore_barrier()

    def _ridx(c):
        return ridx_all_v.at[pl.ds(c * CHUNK, CHUNK)]

    def _cidx(c):
        return cidx_all_v.at[pl.ds(c * CHUNK, CHUNK)]

    pltpu.async_copy(y_hbm.at[_ridx(0)], rows_a, sem_a)

    @pl.loop(0, NCHUNK // 2)
    def _(i):
        c0 = 2 * i
        c1 = c0 + 1
        db = pltpu.async_copy(y_hbm.at[_ridx(c1)], rows_b, sem_b)
        pltpu.make_async_copy(y_hbm.at[_ridx(c0)], rows_a, sem_a).wait()
        pltpu.sync_copy(rows_a, acc_sh.at[_cidx(c0)], add=True)
        pltpu.async_copy(y_hbm.at[_ridx(c1 + 1)], rows_a, sem_a)
        db.wait()
        pltpu.sync_copy(rows_b, acc_sh.at[_cidx(c1)], add=True)

    c_last = NCHUNK - 1
    pltpu.make_async_copy(y_hbm.at[_ridx(c_last)], rows_a, sem_a).wait()
    pltpu.sync_copy(rows_a, acc_sh.at[_cidx(c_last)], add=True)

    plsc.subcore_barrier()
    r0 = sid * RSTRIPE
    pltpu.sync_copy(acc_sh.at[pl.ds(r0, RSTRIPE)],
                    out_hbm.at[cid, pl.ds(r0, RSTRIPE)])


@functools.cache
def _get_scatter():
    mesh = plsc.VectorSubcoreMesh(core_axis_name="core",
                                  subcore_axis_name="subcore")
    return pl.kernel(
        _scatter_body,
        out_type=jax.ShapeDtypeStruct((NC, NPADR, D), jnp.float32),
        mesh=mesh,
        scratch_types=[
            pltpu.VMEM((EPW,), jnp.int32),
            pltpu.VMEM((EPW,), jnp.int32),
            pltpu.VMEM((CHUNK, D), jnp.float32),
            pltpu.VMEM((CHUNK, D), jnp.float32),
            pltpu.VMEM_SHARED((NPADR, D), jnp.float32),
            pltpu.SemaphoreType.DMA,
            pltpu.SemaphoreType.DMA,
        ],
    )


BLK = 1000
GRID = N // BLK


def _dis_block(deg_ref):
    deg = deg_ref[:, 0] + deg_ref[:, 1] + 1.0
    return lax.rsqrt(deg)[:, None]


def _tck1a_body(x_ref, embw_ref, h0_ref):
    h0_ref[...] = jnp.dot(x_ref[...], embw_ref[...].T,
                          preferred_element_type=jnp.float32)


def _tck1b_body(h0_ref, lin_ref, deg_ref, y1_ref):
    ax = jnp.dot(h0_ref[...], lin_ref[...].T,
                 preferred_element_type=jnp.float32)
    y1_ref[...] = _dis_block(deg_ref) * ax


def _tck2_body(h_ref, y_ref, acc_ref, deg_ref, w_ref, b_ref, lin2_ref,
               h1_ref, y2_ref):
    dis = _dis_block(deg_ref)
    h = h_ref[...]
    w = w_ref[...]
    weff = w - w.T - GAMMA * jnp.eye(D, dtype=jnp.float32)
    aggr = dis * (acc_ref[0] + acc_ref[1] + y_ref[...])
    xn = jnp.dot(h, weff, preferred_element_type=jnp.float32) + aggr + b_ref[...]
    h1 = h + EPSILON * jnp.tanh(xn)
    h1_ref[...] = h1
    y2_ref[...] = dis * jnp.dot(h1, lin2_ref[...].T,
                                preferred_element_type=jnp.float32)


def _tck3_body(h_ref, y_ref, acc_ref, deg_ref, w_ref, b_ref, finw_ref,
               finb_ref, emb_ref, out_ref):
    dis = _dis_block(deg_ref)
    h = h_ref[...]
    w = w_ref[...]
    weff = w - w.T - GAMMA * jnp.eye(D, dtype=jnp.float32)
    aggr = dis * (acc_ref[0] + acc_ref[1] + y_ref[...])
    xn = jnp.dot(h, weff, preferred_element_type=jnp.float32) + aggr + b_ref[...]
    h2 = h + EPSILON * jnp.tanh(xn)
    emb_ref[...] = h2
    out_ref[...] = (jnp.dot(h2, finw_ref[...].T,
                            preferred_element_type=jnp.float32)
                    + finb_ref[...])


def _rows(i):
    return (i, 0)


_mat_spec = pl.BlockSpec((BLK, D), _rows)
_full = lambda *s: pl.BlockSpec(s, lambda i: tuple(0 for _ in s))
_deg_spec = pl.BlockSpec((BLK, NC), lambda i: (i, 0))
_acc_spec = pl.BlockSpec((NC, BLK, D), lambda i: (0, i, 0))

_tck1a = pl.pallas_call(
    _tck1a_body,
    grid=(GRID,),
    in_specs=[_mat_spec, _full(D, D)],
    out_specs=_mat_spec,
    out_shape=jax.ShapeDtypeStruct((N, D), jnp.float32),
)

_tck1b = pl.pallas_call(
    _tck1b_body,
    grid=(GRID,),
    in_specs=[_mat_spec, _full(D, D), _deg_spec],
    out_specs=_mat_spec,
    out_shape=jax.ShapeDtypeStruct((N, D), jnp.float32),
)

_tck2 = pl.pallas_call(
    _tck2_body,
    grid=(GRID,),
    in_specs=[_mat_spec, _mat_spec, _acc_spec, _deg_spec,
              _full(D, D), _full(1, D), _full(D, D)],
    out_specs=[_mat_spec, _mat_spec],
    out_shape=[jax.ShapeDtypeStruct((N, D), jnp.float32)] * 2,
)

_tck3 = pl.pallas_call(
    _tck3_body,
    grid=(GRID,),
    in_specs=[_mat_spec, _mat_spec, _acc_spec, _deg_spec,
              _full(D, D), _full(1, D), _full(OUT, D), _full(1, OUT)],
    out_specs=[_mat_spec, pl.BlockSpec((BLK, OUT), _rows)],
    out_shape=[jax.ShapeDtypeStruct((N, D), jnp.float32),
               jax.ShapeDtypeStruct((N, OUT), jnp.float32)],
)


def kernel(x, edge_index, emb_W, conv1_W, conv1_b, conv1_lin,
           conv2_W, conv2_b, conv2_lin, fin_W, fin_b):
    row = edge_index[0]
    col = edge_index[1]
    row3 = row.reshape(NW, NCHUNK, CHUNK)
    degp = _get_hist()(row3).T
    h0 = _tck1a(x, emb_W)
    y1 = _tck1b(h0, conv1_lin, degp)
    acc1 = _get_scatter()(y1, row, col)
    h1, y2 = _tck2(h0, y1, acc1, degp, conv1_W, conv1_b.reshape(1, D),
                   conv2_lin)
    acc2 = _get_scatter()(y2, row, col)
    emb, out = _tck3(h1, y2, acc2, degp, conv2_W, conv2_b.reshape(1, D),
                     fin_W, fin_b.reshape(1, OUT))
    return emb, out


# speedup vs baseline: 1.0242x; 1.0242x over previous
import functools

import jax
import jax.numpy as jnp
from jax import lax
from jax.experimental import pallas as pl
from jax.experimental.pallas import tpu as pltpu
from jax.experimental.pallas import tpu_sc as plsc

GAMMA = 0.1
EPSILON = 0.1
N, E, D, OUT = 10000, 320000, 128, 40

NC, NS = 2, 16
NW = NC * NS
EPW = E // NW
CHUNK = 80
NCHUNK = EPW // CHUNK
NPADH = 10240
HSTRIPE = NPADH // NS
NPADR = 10240
RSTRIPE = NPADR // NS

def _hist_body(row3_hbm, out_hbm, idx_all_v, ones_v, zb_v, hist_sh, sem):
    cid = lax.axis_index("core")
    sid = lax.axis_index("subcore")
    wid = sid * NC + cid

    @pl.loop(0, HSTRIPE // 16)
    def _(i):
        zb_v[pl.ds(i * 16, 16)] = jnp.zeros((16,), jnp.float32)

    @pl.loop(0, CHUNK // 16)
    def _(i):
        ones_v[pl.ds(i * 16, 16)] = jnp.ones((16,), jnp.float32)

    pltpu.async_copy(row3_hbm.at[wid], idx_all_v, sem)
    pltpu.sync_copy(zb_v, hist_sh.at[pl.ds(sid * HSTRIPE, HSTRIPE)])
    pltpu.make_async_copy(row3_hbm.at[wid], idx_all_v, sem).wait()
    plsc.subcore_barrier()

    @pl.loop(0, NCHUNK)
    def _(c):
        pltpu.sync_copy(ones_v, hist_sh.at[idx_all_v.at[c]], add=True)

    plsc.subcore_barrier()
    pltpu.sync_copy(hist_sh.at[pl.ds(sid * HSTRIPE, HSTRIPE)],
                    out_hbm.at[cid, pl.ds(sid * HSTRIPE, HSTRIPE)])


@functools.cache
def _get_hist():
    mesh = plsc.VectorSubcoreMesh(core_axis_name="core",
                                  subcore_axis_name="subcore")
    return pl.kernel(
        _hist_body,
        out_type=jax.ShapeDtypeStruct((NC, NPADH), jnp.float32),
        mesh=mesh,
        scratch_types=[
            pltpu.VMEM((NCHUNK, CHUNK), jnp.int32),
            pltpu.VMEM((CHUNK,), jnp.float32),
            pltpu.VMEM((HSTRIPE,), jnp.float32),
            pltpu.VMEM_SHARED((NPADH,), jnp.float32),
            pltpu.SemaphoreType.DMA,
        ],
    )


def _scatter_body(y_hbm, row_hbm, col_hbm, out_hbm, ridx_all_v, cidx_all_v,
                  rows_a, rows_b, acc_sh, sem_a, sem_b):
    cid = lax.axis_index("core")
    sid = lax.axis_index("subcore")
    wid = sid * NC + cid

    @pl.loop(0, CHUNK)
    def _(r):
        @pl.loop(0, D // 16)
        def _(c):
            rows_a[r, pl.ds(c * 16, 16)] = jnp.zeros((16,), jnp.float32)

    pltpu.async_copy(row_hbm.at[pl.ds(wid * EPW, EPW)], ridx_all_v, sem_b)
    pltpu.async_copy(col_hbm.at[pl.ds(wid * EPW, EPW)], cidx_all_v, sem_b)

    @pl.loop(0, RSTRIPE // CHUNK)
    def _(j):
        pltpu.async_copy(rows_a, acc_sh.at[pl.ds(sid * RSTRIPE + j * CHUNK,
                                                 CHUNK)], sem_a)

    @pl.loop(0, RSTRIPE // CHUNK)
    def _(j):
        pltpu.make_async_copy(rows_a,
                              acc_sh.at[pl.ds(sid * RSTRIPE + j * CHUNK,
                                              CHUNK)], sem_a).wait()
    pltpu.make_async_copy(row_hbm.at[pl.ds(wid * EPW, EPW)], ridx_all_v,
                          sem_b).wait()
    pltpu.make_async_copy(col_hbm.at[pl.ds(wid * EPW, EPW)], cidx_all_v,
                          sem_b).wait()
    plsc.subcore_barrier()

    def _ridx(c):
        return ridx_all_v.at[pl.ds(c * CHUNK, CHUNK)]

    def _cidx(c):
        return cidx_all_v.at[pl.ds(c * CHUNK, CHUNK)]

    pltpu.async_copy(y_hbm.at[_ridx(0)], rows_a, sem_a)

    @pl.loop(0, NCHUNK // 2)
    def _(i):
        c0 = 2 * i
        c1 = c0 + 1
        db = pltpu.async_copy(y_hbm.at[_ridx(c1)], rows_b, sem_b)
        pltpu.make_async_copy(y_hbm.at[_ridx(c0)], rows_a, sem_a).wait()
        pltpu.sync_copy(rows_a, acc_sh.at[_cidx(c0)], add=True)
        pltpu.async_copy(y_hbm.at[_ridx(c1 + 1)], rows_a, sem_a)
        db.wait()
        pltpu.sync_copy(rows_b, acc_sh.at[_cidx(c1)], add=True)

    c_last = NCHUNK - 1
    pltpu.make_async_copy(y_hbm.at[_ridx(c_last)], rows_a, sem_a).wait()
    pltpu.sync_copy(rows_a, acc_sh.at[_cidx(c_last)], add=True)

    plsc.subcore_barrier()
    r0 = sid * RSTRIPE
    pltpu.sync_copy(acc_sh.at[pl.ds(r0, RSTRIPE)],
                    out_hbm.at[cid, pl.ds(r0, RSTRIPE)])


@functools.cache
def _get_scatter():
    mesh = plsc.VectorSubcoreMesh(core_axis_name="core",
                                  subcore_axis_name="subcore")
    return pl.kernel(
        _scatter_body,
        out_type=jax.ShapeDtypeStruct((NC, NPADR, D), jnp.float32),
        mesh=mesh,
        scratch_types=[
            pltpu.VMEM((EPW,), jnp.int32),
            pltpu.VMEM((EPW,), jnp.int32),
            pltpu.VMEM((CHUNK, D), jnp.float32),
            pltpu.VMEM((CHUNK, D), jnp.float32),
            pltpu.VMEM_SHARED((NPADR, D), jnp.float32),
            pltpu.SemaphoreType.DMA,
            pltpu.SemaphoreType.DMA,
        ],
    )


BLK = 1024
GRID = pl.cdiv(N, BLK)


def _dis_block(deg_ref):
    sl = pl.ds(pl.program_id(0) * BLK, BLK)
    deg = deg_ref[0, sl] + deg_ref[1, sl] + 1.0
    return lax.rsqrt(deg)[:, None]


def _tck1a_body(x_ref, embw_ref, h0_ref):
    h0_ref[...] = jnp.dot(x_ref[...], embw_ref[...].T,
                          preferred_element_type=jnp.float32)


def _tck1b_body(h0_ref, lin_ref, deg_ref, y1_ref):
    ax = jnp.dot(h0_ref[...], lin_ref[...].T,
                 preferred_element_type=jnp.float32)
    y1_ref[...] = _dis_block(deg_ref) * ax


def _tck2_body(h_ref, y_ref, acc_ref, deg_ref, w_ref, b_ref, lin2_ref,
               h1_ref, y2_ref):
    dis = _dis_block(deg_ref)
    h = h_ref[...]
    w = w_ref[...]
    weff = w - w.T - GAMMA * jnp.eye(D, dtype=jnp.float32)
    aggr = dis * (acc_ref[0] + acc_ref[1] + y_ref[...])
    xn = jnp.dot(h, weff, preferred_element_type=jnp.float32) + aggr + b_ref[...]
    h1 = h + EPSILON * jnp.tanh(xn)
    h1_ref[...] = h1
    y2_ref[...] = dis * jnp.dot(h1, lin2_ref[...].T,
                                preferred_element_type=jnp.float32)


def _tck3_body(h_ref, y_ref, acc_ref, deg_ref, w_ref, b_ref, finw_ref,
               finb_ref, emb_ref, out_ref):
    dis = _dis_block(deg_ref)
    h = h_ref[...]
    w = w_ref[...]
    weff = w - w.T - GAMMA * jnp.eye(D, dtype=jnp.float32)
    aggr = dis * (acc_ref[0] + acc_ref[1] + y_ref[...])
    xn = jnp.dot(h, weff, preferred_element_type=jnp.float32) + aggr + b_ref[...]
    h2 = h + EPSILON * jnp.tanh(xn)
    emb_ref[...] = h2
    out_ref[...] = (jnp.dot(h2, finw_ref[...].T,
                            preferred_element_type=jnp.float32)
                    + finb_ref[...])


def _rows(i):
    return (i, 0)


_mat_spec = pl.BlockSpec((BLK, D), _rows)
_full = lambda *s: pl.BlockSpec(s, lambda i: tuple(0 for _ in s))
_deg_spec = pl.BlockSpec((NC, NPADH), lambda i: (0, 0))
_acc_spec = pl.BlockSpec((NC, BLK, D), lambda i: (0, i, 0))

_tck1a = pl.pallas_call(
    _tck1a_body,
    grid=(GRID,),
    in_specs=[_mat_spec, _full(D, D)],
    out_specs=_mat_spec,
    out_shape=jax.ShapeDtypeStruct((N, D), jnp.float32),
)

_tck1b = pl.pallas_call(
    _tck1b_body,
    grid=(GRID,),
    in_specs=[_mat_spec, _full(D, D), _deg_spec],
    out_specs=_mat_spec,
    out_shape=jax.ShapeDtypeStruct((N, D), jnp.float32),
)

_tck2 = pl.pallas_call(
    _tck2_body,
    grid=(GRID,),
    in_specs=[_mat_spec, _mat_spec, _acc_spec, _deg_spec,
              _full(D, D), _full(1, D), _full(D, D)],
    out_specs=[_mat_spec, _mat_spec],
    out_shape=[jax.ShapeDtypeStruct((N, D), jnp.float32)] * 2,
)

_tck3 = pl.pallas_call(
    _tck3_body,
    grid=(GRID,),
    in_specs=[_mat_spec, _mat_spec, _acc_spec, _deg_spec,
              _full(D, D), _full(1, D), _full(OUT, D), _full(1, OUT)],
    out_specs=[_mat_spec, pl.BlockSpec((BLK, OUT), _rows)],
    out_shape=[jax.ShapeDtypeStruct((N, D), jnp.float32),
               jax.ShapeDtypeStruct((N, OUT), jnp.float32)],
)


def kernel(x, edge_index, emb_W, conv1_W, conv1_b, conv1_lin,
           conv2_W, conv2_b, conv2_lin, fin_W, fin_b):
    row = edge_index[0]
    col = edge_index[1]
    row3 = row.reshape(NW, NCHUNK, CHUNK)
    degp = _get_hist()(row3)
    h0 = _tck1a(x, emb_W)
    y1 = _tck1b(h0, conv1_lin, degp)
    acc1 = _get_scatter()(y1, row, col)
    h1, y2 = _tck2(h0, y1, acc1, degp, conv1_W, conv1_b.reshape(1, D),
                   conv2_lin)
    acc2 = _get_scatter()(y2, row, col)
    emb, out = _tck3(h1, y2, acc2, degp, conv2_W, conv2_b.reshape(1, D),
                     fin_W, fin_b.reshape(1, OUT))
    return emb, out
